# Initial kernel scaffold; baseline (speedup 1.0000x reference)
import jax
import jax.numpy as jnp
from jax.experimental import pallas as pl


def kernel(x, W):
    raise NotImplementedError("write your pallas kernel here")


# speedup vs baseline: 1.1914x; 1.1914x over previous
import functools

import jax
import jax.numpy as jnp
from jax.experimental import pallas as pl

HIDDEN = 4096
N_EXPERTS = 64
TOP_K = 8
BT = 512


def _gate_block(x_ref, w_ref, vals_ref, idx_ref):
    logits = jax.lax.dot_general(
        x_ref[...].astype(jnp.bfloat16), w_ref[...].astype(jnp.bfloat16),
        dimension_numbers=(((1,), (1,)), ((), ())),
        preferred_element_type=jnp.float32,
    )
    m = jnp.max(logits, axis=1, keepdims=True)
    e = jnp.exp(logits - m)
    p = e / jnp.sum(e, axis=1, keepdims=True)

    iota = jax.lax.broadcasted_iota(jnp.int32, p.shape, 1)
    for k in range(TOP_K):
        v = jnp.max(p, axis=1, keepdims=True)
        cand = jnp.where(p == v, iota, N_EXPERTS)
        ix = jnp.min(cand, axis=1, keepdims=True)
        vals_ref[:, k] = v[:, 0]
        idx_ref[:, k] = ix[:, 0]
        p = jnp.where(iota == ix, -1.0, p)


@jax.jit
def kernel(x, W):
    tokens = x.shape[0]
    grid = (tokens // BT,)
    vals, idx = pl.pallas_call(
        _gate_block,
        grid=grid,
        in_specs=[
            pl.BlockSpec((BT, HIDDEN), lambda i: (i, 0)),
            pl.BlockSpec((N_EXPERTS, HIDDEN), lambda i: (0, 0)),
        ],
        out_specs=[
            pl.BlockSpec((BT, TOP_K), lambda i: (i, 0)),
            pl.BlockSpec((BT, TOP_K), lambda i: (i, 0)),
        ],
        out_shape=[
            jax.ShapeDtypeStruct((tokens, TOP_K), jnp.float32),
            jax.ShapeDtypeStruct((tokens, TOP_K), jnp.int32),
        ],
    )(x, W)
    return vals, idx


# speedup vs baseline: 1.3879x; 1.1649x over previous
import functools

import jax
import jax.numpy as jnp
from jax.experimental import pallas as pl

HIDDEN = 4096
N_EXPERTS = 64
TOP_K = 8
BT = 1024


def _gate_block(x_ref, w_ref, vals_ref, idx_ref):
    logits = jax.lax.dot_general(
        x_ref[...].astype(jnp.bfloat16), w_ref[...].astype(jnp.bfloat16),
        dimension_numbers=(((1,), (1,)), ((), ())),
        preferred_element_type=jnp.float32,
    )
    m = jnp.max(logits, axis=1, keepdims=True)
    e = jnp.exp(logits - m)
    p = e / jnp.sum(e, axis=1, keepdims=True)

    iota = jax.lax.broadcasted_iota(jnp.int32, p.shape, 1)
    for k in range(TOP_K):
        v = jnp.max(p, axis=1, keepdims=True)
        cand = jnp.where(p == v, iota, N_EXPERTS)
        ix = jnp.min(cand, axis=1, keepdims=True)
        vals_ref[:, k] = v[:, 0]
        idx_ref[:, k] = ix[:, 0]
        p = jnp.where(iota == ix, -1.0, p)


@jax.jit
def kernel(x, W):
    tokens = x.shape[0]
    grid = (tokens // BT,)
    vals, idx = pl.pallas_call(
        _gate_block,
        grid=grid,
        in_specs=[
            pl.BlockSpec((BT, HIDDEN), lambda i: (i, 0)),
            pl.BlockSpec((N_EXPERTS, HIDDEN), lambda i: (0, 0)),
        ],
        out_specs=[
            pl.BlockSpec((BT, TOP_K), lambda i: (i, 0)),
            pl.BlockSpec((BT, TOP_K), lambda i: (i, 0)),
        ],
        out_shape=[
            jax.ShapeDtypeStruct((tokens, TOP_K), jnp.float32),
            jax.ShapeDtypeStruct((tokens, TOP_K), jnp.int32),
        ],
    )(x, W)
    return vals, idx


# speedup vs baseline: 1.5034x; 1.0832x over previous
import functools

import jax
import jax.numpy as jnp
from jax.experimental import pallas as pl

HIDDEN = 4096
N_EXPERTS = 64
TOP_K = 8
BT = 1024


def _gate_block(x_ref, w_ref, vals_ref, idx_ref):
    logits = jax.lax.dot_general(
        x_ref[...].astype(jnp.bfloat16), w_ref[...].astype(jnp.bfloat16),
        dimension_numbers=(((1,), (1,)), ((), ())),
        preferred_element_type=jnp.float32,
    )
    m = jnp.max(logits, axis=1, keepdims=True)
    e = jnp.exp(logits - m)
    s = jnp.sum(e, axis=1, keepdims=True)

    iota = jax.lax.broadcasted_iota(jnp.int32, e.shape, 1).astype(jnp.float32)
    for k in range(TOP_K):
        v = jnp.max(e, axis=1, keepdims=True)
        cand = jnp.where(e == v, iota, float(N_EXPERTS))
        ix = jnp.min(cand, axis=1, keepdims=True)
        vals_ref[:, k] = (v / s)[:, 0]
        idx_ref[:, k] = ix[:, 0].astype(jnp.int32)
        e = jnp.where(iota == ix, -1.0, e)


@jax.jit
def kernel(x, W):
    tokens = x.shape[0]
    grid = (tokens // BT,)
    vals, idx = pl.pallas_call(
        _gate_block,
        grid=grid,
        in_specs=[
            pl.BlockSpec((BT, HIDDEN), lambda i: (i, 0)),
            pl.BlockSpec((N_EXPERTS, HIDDEN), lambda i: (0, 0)),
        ],
        out_specs=[
            pl.BlockSpec((BT, TOP_K), lambda i: (i, 0)),
            pl.BlockSpec((BT, TOP_K), lambda i: (i, 0)),
        ],
        out_shape=[
            jax.ShapeDtypeStruct((tokens, TOP_K), jnp.float32),
            jax.ShapeDtypeStruct((tokens, TOP_K), jnp.int32),
        ],
    )(x, W)
    return vals, idx
